# baseline (speedup 1.0000x reference)
import functools

import jax
import jax.numpy as jnp
from jax import lax
from jax.experimental import pallas as pl
from jax.experimental.pallas import tpu as pltpu
from jax.experimental.pallas import tpu_sc as plsc

D = 64
W = 2 * D
SCALE = float(D) ** 0.5

NC = 2
NS = 16
NW = NC * NS
L = 16
G = 128
CHUNK = 256


def _body(n_chunks, table_hbm, idx_hbm, out_hbm,
          idx_v, rows_v, obuf_v, sem_rows, sem_out):
    wid = lax.axis_index("s") * NC + lax.axis_index("c")
    b_per_w = CHUNK * n_chunks
    base = wid * b_per_w
    K = CHUNK // G

    row0 = pl.multiple_of(base // G, 8)
    pltpu.sync_copy(idx_hbm.at[pl.ds(row0, b_per_w // G), :], idx_v)

    def gather(buf, g, do_start):
        for k in range(K):
            cp = pltpu.make_async_copy(
                table_hbm.at[idx_v.at[g * K + k]],
                rows_v.at[buf, pl.ds(k * G, G), :], sem_rows)
            if do_start:
                cp.start()
            else:
                cp.wait()

    def out_copy(g, buf):
        r0 = pl.multiple_of((base + g * CHUNK) // 2, CHUNK // 2)
        return pltpu.make_async_copy(
            obuf_v.at[buf], out_hbm.at[pl.ds(r0, CHUNK // 2)], sem_out)

    def pack_rows(buf):
        rb = rows_v.at[buf]
        ob = obuf_v.at[buf]

        def row_fn(k, _):
            for j in range(D // L):
                ob[k, pl.ds(j * L, L)] = rb[2 * k, pl.ds(j * L, L)]
                ob[k, pl.ds(D + j * L, L)] = rb[2 * k + 1, pl.ds(j * L, L)]
            return 0
        lax.fori_loop(0, CHUNK // 2, row_fn, 0, unroll=4)

    gather(0, 0, True)

    def step(g, _):
        buf = lax.rem(g, 2)
        nxt = 1 - buf

        @pl.when(g >= 1)
        def _():
            out_copy(g - 1, nxt).wait()

        gather(buf, g, False)

        @pl.when(g + 1 < n_chunks)
        def _():
            gather(nxt, g + 1, True)

        pack_rows(buf)
        out_copy(g, buf).start()
        return 0

    lax.fori_loop(0, n_chunks, step, 0)
    out_copy(n_chunks - 1, lax.rem(n_chunks - 1, 2)).wait()


PREP_C = 16384


def _prep_body(tt_ref, out_ref):
    blk = tt_ref[...]
    out_ref[:, :D] = jnp.transpose(blk, (1, 0)) * SCALE
    out_ref[:, D:] = jnp.zeros((PREP_C, D), jnp.float32)


@jax.jit
def _prep(tt):
    v = tt.shape[1]
    return pl.pallas_call(
        _prep_body,
        grid=(pl.cdiv(v, PREP_C),),
        in_specs=[pl.BlockSpec((D, PREP_C), lambda i: (0, i))],
        out_specs=pl.BlockSpec((PREP_C, W), lambda i: (i, 0)),
        out_shape=jax.ShapeDtypeStruct((v, W), jnp.float32),
    )(tt)


@functools.partial(jax.jit, static_argnames=("n_chunks",))
def _embed_sc(idx2, table_wide, n_chunks):
    b_total = CHUNK * n_chunks * NW
    mesh = plsc.VectorSubcoreMesh(core_axis_name="c", subcore_axis_name="s")
    run = pl.kernel(
        functools.partial(_body, n_chunks),
        out_type=jax.ShapeDtypeStruct((b_total // 2, W), jnp.float32),
        mesh=mesh,
        scratch_types=[
            pltpu.VMEM((CHUNK * n_chunks // G, G), jnp.int32),
            pltpu.VMEM((2, CHUNK, W), jnp.float32),
            pltpu.VMEM((2, CHUNK // 2, W), jnp.float32),
            pltpu.SemaphoreType.DMA,
            pltpu.SemaphoreType.DMA,
        ],
        compiler_params=pltpu.CompilerParams(use_tc_tiling_on_sc=True,
                                             needs_layout_passes=False),
    )
    return run(table_wide, idx2)


def kernel(inputs, lookup_table):
    B, T = inputs.shape
    idx2 = inputs.reshape(B * T // G, G).astype(jnp.int32)
    table_wide = _prep(lookup_table.T)
    b_per_w = (B * T) // NW
    out = _embed_sc(idx2, table_wide, b_per_w // CHUNK)
    return out.reshape(B * T, D).reshape(B, T, D)


# speedup vs baseline: 1.5938x; 1.5938x over previous
import functools

import jax
import jax.numpy as jnp
from jax import lax
from jax.experimental import pallas as pl
from jax.experimental.pallas import tpu as pltpu
from jax.experimental.pallas import tpu_sc as plsc

D = 64
W = 2 * D
SCALE = float(D) ** 0.5

NC = 2
NS = 16
NW = NC * NS
L = 16
G = 128
CHUNK = 256


def _body(n_chunks, table_hbm, idx_hbm, out_hbm,
          idx_v, rows_v, sem_rows, sem_out):
    wid = lax.axis_index("s") * NC + lax.axis_index("c")
    b_per_w = CHUNK * n_chunks
    base = wid * b_per_w
    K = CHUNK // G

    row0 = pl.multiple_of(base // G, 8)
    pltpu.sync_copy(idx_hbm.at[pl.ds(row0, b_per_w // G), :], idx_v)

    def gather(buf, g, do_start):
        for k in range(K):
            cp = pltpu.make_async_copy(
                table_hbm.at[idx_v.at[g * K + k]],
                rows_v.at[buf, pl.ds(k * G, G), :], sem_rows)
            if do_start:
                cp.start()
            else:
                cp.wait()

    def out_copy(g, buf):
        return pltpu.make_async_copy(
            rows_v.at[buf],
            out_hbm.at[pl.ds(pl.multiple_of(base + g * CHUNK, CHUNK), CHUNK)],
            sem_out)

    gather(0, 0, True)

    def step(g, _):
        buf = lax.rem(g, 2)
        nxt = 1 - buf

        @pl.when(g >= 1)
        def _():
            out_copy(g - 1, nxt).wait()

        gather(buf, g, False)

        @pl.when(g + 1 < n_chunks)
        def _():
            gather(nxt, g + 1, True)

        out_copy(g, buf).start()
        return 0

    lax.fori_loop(0, n_chunks, step, 0)
    out_copy(n_chunks - 1, lax.rem(n_chunks - 1, 2)).wait()


PREP_C = 32768


def _prep_body(tt_ref, out_ref):
    blk = tt_ref[...]
    out_ref[:, :D] = jnp.transpose(blk, (1, 0)) * SCALE
    out_ref[:, D:] = jnp.zeros((PREP_C, D), jnp.float32)


@jax.jit
def _prep(tt):
    v = tt.shape[1]
    return pl.pallas_call(
        _prep_body,
        grid=(pl.cdiv(v, PREP_C),),
        in_specs=[pl.BlockSpec((D, PREP_C), lambda i: (0, i))],
        out_specs=pl.BlockSpec((PREP_C, W), lambda i: (i, 0)),
        out_shape=jax.ShapeDtypeStruct((v, W), jnp.float32),
    )(tt)


@functools.partial(jax.jit, static_argnames=("n_chunks",))
def _embed_sc(idx2, table_wide, n_chunks):
    b_total = CHUNK * n_chunks * NW
    mesh = plsc.VectorSubcoreMesh(core_axis_name="c", subcore_axis_name="s")
    run = pl.kernel(
        functools.partial(_body, n_chunks),
        out_type=jax.ShapeDtypeStruct((b_total, W), jnp.float32),
        mesh=mesh,
        scratch_types=[
            pltpu.VMEM((CHUNK * n_chunks // G, G), jnp.int32),
            pltpu.VMEM((2, CHUNK, W), jnp.float32),
            pltpu.SemaphoreType.DMA,
            pltpu.SemaphoreType.DMA,
        ],
        compiler_params=pltpu.CompilerParams(use_tc_tiling_on_sc=True,
                                             needs_layout_passes=False),
    )
    return run(table_wide, idx2)


def kernel(inputs, lookup_table):
    B, T = inputs.shape
    idx2 = inputs.reshape(B * T // G, G).astype(jnp.int32)
    table_wide = _prep(lookup_table.T)
    b_per_w = (B * T) // NW
    out = _embed_sc(idx2, table_wide, b_per_w // CHUNK)
    return out[:, :D].reshape(B, T, D)
